# baseline (speedup 1.0000x reference)
import math

import jax
import jax.numpy as jnp
from jax import lax
from jax.experimental import pallas as pl
from jax.experimental.pallas import tpu as pltpu
from jax.experimental.pallas import tpu_sc as plsc

EMB = 64
SCALE = math.sqrt(EMB)
SEQ = 200
NBUF = 2
SPLITS = ((0, 128), (128, 72))
IDXPAD = 16
LANES = 128


def _transpose_scale(table_t, vocab):
    info = plsc.get_sparse_core_info()
    n_workers = info.num_cores * info.num_subcores
    n_full = vocab // LANES
    rem = vocab - n_full * LANES
    per_w = n_full // n_workers
    extra = n_full - per_w * n_workers
    mesh = plsc.VectorSubcoreMesh(core_axis_name="c", subcore_axis_name="s")

    def body(tt_hbm, out_hbm, in_vm, out_vm, rem_vm, si, so):
        wid = lax.axis_index("s") * info.num_cores + lax.axis_index("c")
        nblk = per_w + jnp.where(wid < extra, 1, 0)

        def blk_id(k):
            return wid + k * n_workers

        def fire(k, b):
            pltpu.async_copy(
                tt_hbm.at[:, pl.ds(blk_id(k) * LANES, LANES)], in_vm[b], si[b]
            )

        def wait_in(b):
            pltpu.make_async_copy(
                tt_hbm.at[:, pl.ds(0, LANES)], in_vm[b], si[b]
            ).wait()

        def store(k, b):
            pltpu.async_copy(
                out_vm[b], out_hbm.at[pl.ds(blk_id(k) * (LANES // 2), LANES // 2)], so[b]
            )

        def wait_store(b):
            pltpu.make_async_copy(
                out_vm[b], out_hbm.at[pl.ds(0, LANES // 2)], so[b]
            ).wait()

        lanes16 = lax.iota(jnp.int32, 16)

        def transpose_block(b, width):
            def row_body(jj, c):
                for par in range(2):
                    t_local = 2 * jj + par
                    col = jnp.full((16,), t_local, jnp.int32)
                    for q in range(EMB // 16):
                        v = plsc.load_gather(in_vm[b], [q * 16 + lanes16, col])
                        out_vm[b][jj, pl.ds(par * EMB + q * 16, 16)] = v * SCALE
                return c

            lax.fori_loop(0, width // 2, row_body, 0)

        fire(0, 0)

        def step(k2, carry):
            for b in range(NBUF):
                k = k2 * NBUF + b
                nb = (b + 1) % NBUF

                @pl.when(jnp.logical_and(k >= 1, k < nblk))
                def _():
                    wait_store(nb)

                @pl.when(k + 1 < nblk)
                def _():
                    fire(k + 1, nb)

                @pl.when(k < nblk)
                def _():
                    wait_in(b)
                    transpose_block(b, LANES)
                    store(k, b)
            return carry

        lax.fori_loop(0, (per_w + 1 + NBUF - 1) // NBUF, step, 0)

        @pl.when(nblk % 2 == 1)
        def _():
            wait_store(0)

        @pl.when(nblk % 2 == 0)
        def _():
            wait_store(1)

        if rem:
            @pl.when(wid == 0)
            def _():
                pltpu.sync_copy(tt_hbm.at[:, pl.ds(n_full * LANES, rem)], rem_vm)
                def row_body(jj, c):
                    for par in range(2):
                        t_local = 2 * jj + par
                        col = jnp.full((16,), t_local, jnp.int32)
                        for q in range(EMB // 16):
                            v = plsc.load_gather(rem_vm, [q * 16 + lanes16, col])
                            out_vm[0][jj, pl.ds(par * EMB + q * 16, 16)] = v * SCALE
                    return c
                lax.fori_loop(0, rem // 2, row_body, 0)
                pltpu.sync_copy(
                    out_vm[0].at[pl.ds(0, rem // 2)],
                    out_hbm.at[pl.ds(n_full * (LANES // 2), rem // 2)],
                )

    return pl.kernel(
        body,
        out_type=jax.ShapeDtypeStruct((vocab // 2, 2 * EMB), jnp.float32),
        mesh=mesh,
        scratch_types=[
            [pltpu.VMEM((EMB, LANES), jnp.float32) for _ in range(NBUF)],
            [pltpu.VMEM((LANES // 2, 2 * EMB), jnp.float32) for _ in range(NBUF)],
            pltpu.VMEM((EMB, EMB), jnp.float32),
            [pltpu.SemaphoreType.DMA for _ in range(NBUF)],
            [pltpu.SemaphoreType.DMA for _ in range(NBUF)],
        ],
        compiler_params=pltpu.CompilerParams(needs_layout_passes=False),
    )(table_t)


def _gather(tokens_flat, table2, nbatch):
    B = tokens_flat.shape[0]
    info = plsc.get_sparse_core_info()
    n_workers = info.num_cores * info.num_subcores
    rows_per_w = nbatch // n_workers
    toks_per_w = rows_per_w * SEQ
    mesh = plsc.VectorSubcoreMesh(core_axis_name="c", subcore_axis_name="s")

    def body(tokens_hbm, table_hbm, out_hbm, idx_all, idx2, rows, rows_o, sg, ss):
        wid = lax.axis_index("s") * info.num_cores + lax.axis_index("c")
        wrow0 = wid * rows_per_w

        pltpu.sync_copy(
            tokens_hbm.at[pl.ds(wid * toks_per_w, toks_per_w)],
            idx_all.at[pl.ds(0, toks_per_w)],
        )

        def fire(ci, b):
            base = ci * SEQ
            for j in range((SEQ + 15) // 16):
                sl = pl.ds(j * 16, 16)
                idx2[b][sl] = lax.shift_right_logical(idx_all[pl.ds(base + j * 16, 16)], 1)
            for (off, n) in SPLITS:
                pltpu.async_copy(
                    table_hbm.at[idx2[b].at[pl.ds(off, n)]],
                    rows[b].at[pl.ds(off, n)],
                    sg[b],
                )

        def wait_gather(b):
            for (off, n) in SPLITS:
                pltpu.make_async_copy(
                    table_hbm.at[idx2[b].at[pl.ds(off, n)]],
                    rows[b].at[pl.ds(off, n)],
                    sg[b],
                ).wait()

        def store(ci, b):
            pltpu.async_copy(rows_o[b], out_hbm.at[wrow0 + ci], ss[b])

        def wait_store(b):
            pltpu.make_async_copy(rows_o[b], out_hbm.at[wrow0], ss[b]).wait()

        def select(ci, b):
            base = ci * SEQ

            def sel_block(k, c):
                r0 = k * 8
                par_vec = (idx_all[pl.ds(base + r0, 16)] & 1) * EMB
                for j in range(8):
                    par = par_vec[j]
                    for q in range(EMB // 16):
                        v = rows[b][r0 + j, pl.ds(par + q * 16, 16)]
                        rows_o[b][r0 + j, pl.ds(q * 16, 16)] = v
                return c

            lax.fori_loop(0, SEQ // 8, sel_block, 0)

        fire(0, 0)

        def step(ci, carry):
            for b in range(NBUF):
                i = ci * NBUF + b
                nb = (b + 1) % NBUF
                @pl.when(i >= 1)
                def _():
                    wait_store(nb)
                @pl.when(i + 1 < rows_per_w)
                def _():
                    fire(i + 1, nb)
                wait_gather(b)
                select(i, b)
                store(i, b)
            return carry

        lax.fori_loop(0, rows_per_w // NBUF, step, 0)
        wait_store((rows_per_w - 1) % NBUF)

    return pl.kernel(
        body,
        out_type=jax.ShapeDtypeStruct((nbatch, SEQ, EMB), jnp.float32),
        mesh=mesh,
        scratch_types=[
            pltpu.VMEM((toks_per_w + IDXPAD,), jnp.int32),
            [pltpu.VMEM((SEQ + IDXPAD,), jnp.int32) for _ in range(NBUF)],
            [pltpu.VMEM((SEQ, 2 * EMB), jnp.float32) for _ in range(NBUF)],
            [pltpu.VMEM((SEQ, EMB), jnp.float32) for _ in range(NBUF)],
            [pltpu.SemaphoreType.DMA for _ in range(NBUF)],
            [pltpu.SemaphoreType.DMA for _ in range(NBUF)],
        ],
    )(tokens_flat, table2)


def kernel(tokens, table):
    nbatch, seq = tokens.shape
    assert seq == SEQ
    vocab = table.shape[0]
    table2 = _transpose_scale(table.T, vocab)
    return _gather(tokens.reshape(nbatch * seq), table2, nbatch)


# speedup vs baseline: 1.4004x; 1.4004x over previous
import math

import jax
import jax.numpy as jnp
from jax import lax
from jax.experimental import pallas as pl
from jax.experimental.pallas import tpu as pltpu
from jax.experimental.pallas import tpu_sc as plsc

EMB = 64
SCALE = math.sqrt(EMB)
SEQ = 200
NBUF = 4
SPLITS = ((0, 128), (128, 72))
TBLK = 1024


def _tc_transpose_table(table_t, vocab):
    n_blk = (vocab + TBLK - 1) // TBLK

    def body(in_ref, out_ref):
        t = in_ref[...].T * SCALE
        out_ref[...] = jnp.concatenate([t, jnp.zeros_like(t)], axis=1)

    return pl.pallas_call(
        body,
        grid=(n_blk,),
        in_specs=[pl.BlockSpec((EMB, TBLK), lambda c: (0, c))],
        out_specs=pl.BlockSpec((TBLK, 2 * EMB), lambda c: (c, 0)),
        out_shape=jax.ShapeDtypeStruct((vocab, 2 * EMB), jnp.float32),
    )(table_t)


def _sc_gather(tokens_flat, table_lin, nbatch):
    B = tokens_flat.shape[0]
    info = plsc.get_sparse_core_info()
    n_workers = info.num_cores * info.num_subcores
    rows_per_w = nbatch // n_workers
    toks_per_w = rows_per_w * SEQ
    mesh = plsc.VectorSubcoreMesh(core_axis_name="c", subcore_axis_name="s")

    def body(tokens_hbm, table_hbm, out_hbm, idx_all, idx2, rows, sg, ss):
        wid = lax.axis_index("s") * info.num_cores + lax.axis_index("c")
        wrow0 = wid * rows_per_w

        pltpu.sync_copy(
            tokens_hbm.at[pl.ds(wid * toks_per_w, toks_per_w)],
            idx_all.at[pl.ds(0, toks_per_w)],
        )

        def fire(ci, b):
            base = ci * SEQ
            for j in range((SEQ + 15) // 16):
                sl = pl.ds(j * 16, 16)
                idx2[b][sl] = lax.shift_left(idx_all[pl.ds(base + j * 16, 16)], 1)
            for (off, n) in SPLITS:
                pltpu.async_copy(
                    table_hbm.at[idx2[b].at[pl.ds(off, n)]],
                    rows[b].at[pl.ds(off, n)],
                    sg[b],
                )

        def wait_gather(b):
            for (off, n) in SPLITS:
                pltpu.make_async_copy(
                    table_hbm.at[idx2[b].at[pl.ds(off, n)]],
                    rows[b].at[pl.ds(off, n)],
                    sg[b],
                ).wait()

        def store(ci, b):
            pltpu.async_copy(rows[b], out_hbm.at[wrow0 + ci], ss[b])

        def wait_store(b):
            pltpu.make_async_copy(rows[b], out_hbm.at[wrow0], ss[b]).wait()

        for b in range(NBUF - 1):
            fire(b, b)

        def ring_cycle(k, carry):
            for b in range(NBUF):
                ci = k * NBUF + b
                wait_gather(b)
                store(ci, b)
                pb = (b - 1) % NBUF
                @pl.when(ci >= 1)
                def _():
                    wait_store(pb)
                @pl.when(ci + NBUF - 1 < rows_per_w)
                def _():
                    fire(ci + NBUF - 1, pb)
            return carry

        lax.fori_loop(0, rows_per_w // NBUF, ring_cycle, 0)
        wait_store((rows_per_w - 1) % NBUF)

    return pl.kernel(
        body,
        out_type=jax.ShapeDtypeStruct((nbatch, SEQ, EMB), jnp.float32),
        mesh=mesh,
        scratch_types=[
            pltpu.VMEM((toks_per_w + 16,), jnp.int32),
            [pltpu.VMEM((SEQ + 16,), jnp.int32) for _ in range(NBUF)],
            [pltpu.VMEM((SEQ, EMB), jnp.float32) for _ in range(NBUF)],
            [pltpu.SemaphoreType.DMA for _ in range(NBUF)],
            [pltpu.SemaphoreType.DMA for _ in range(NBUF)],
        ],
        compiler_params=pltpu.CompilerParams(use_tc_tiling_on_sc=False),
    )(tokens_flat, table_lin)


def _tc_output_layout(y_lin, nbatch):
    nb = nbatch // 128

    def body(in_ref, out_ref):
        for sj in range(8):
            out_ref[sj] = in_ref[:, 0, sj, :].T.reshape(8, 1, 8, 128)

    return pl.pallas_call(
        body,
        grid=(SEQ // 8, nb),
        in_specs=[pl.BlockSpec((128, 1, 8, EMB), lambda sh, c: (c, sh, 0, 0))],
        out_specs=pl.BlockSpec((8, 8, 1, 8, 128), lambda sh, c: (sh, 0, c, 0, 0)),
        out_shape=jax.ShapeDtypeStruct((SEQ, 8, nb, 8, 128), jnp.float32),
    )(y_lin.reshape(y_lin.shape[0], SEQ // 8, 8, EMB))


def kernel(tokens, table):
    nbatch, seq = tokens.shape
    assert seq == SEQ
    vocab = table.shape[0]
    scratch = _tc_transpose_table(table.T, vocab)
    y = _sc_gather(tokens.reshape(nbatch * seq), scratch.reshape(2 * vocab, EMB), nbatch)
    out5 = _tc_output_layout(y, nbatch)
    return jnp.transpose(out5, (2, 4, 0, 1, 3)).reshape(nbatch, SEQ, EMB)


# speedup vs baseline: 2.0614x; 1.4720x over previous
import math

import jax
import jax.numpy as jnp
from jax import lax
from jax.experimental import pallas as pl
from jax.experimental.pallas import tpu as pltpu
from jax.experimental.pallas import tpu_sc as plsc

EMB = 64
SCALE = math.sqrt(EMB)
SEQ = 200
NBUF = 4
SPLITS = ((0, 128), (128, 72))


def kernel(tokens, table):
    nbatch, seq = tokens.shape
    assert seq == SEQ
    B = nbatch * seq
    info = plsc.get_sparse_core_info()
    n_workers = info.num_cores * info.num_subcores
    rows_per_w = nbatch // n_workers
    toks_per_w = rows_per_w * seq
    mesh = plsc.VectorSubcoreMesh(core_axis_name="c", subcore_axis_name="s")

    def body(tokens_hbm, table_hbm, out_hbm, idx_v, rows, sg, ss):
        wid = lax.axis_index("s") * info.num_cores + lax.axis_index("c")
        wrow0 = wid * rows_per_w

        pltpu.sync_copy(tokens_hbm.at[pl.ds(wid * toks_per_w, toks_per_w)], idx_v)

        def fire(ci, b):
            for (off, n) in SPLITS:
                pltpu.async_copy(
                    table_hbm.at[idx_v.at[pl.ds(ci * SEQ + off, n)]],
                    rows[b].at[pl.ds(off, n)],
                    sg[b],
                )

        def wait_gather(b):
            for (off, n) in SPLITS:
                pltpu.make_async_copy(
                    table_hbm.at[idx_v.at[pl.ds(off, n)]],
                    rows[b].at[pl.ds(off, n)],
                    sg[b],
                ).wait()

        def store(ci, b):
            pltpu.async_copy(rows[b], out_hbm.at[wrow0 + ci], ss[b])

        def wait_store(b):
            pltpu.make_async_copy(rows[b], out_hbm.at[wrow0], ss[b]).wait()

        def scale(b):
            def scale_row(r, c):
                for q in range(EMB // 16):
                    sl = pl.ds(q * 16, 16)
                    rows[b][r, sl] = rows[b][r, sl] * SCALE
                return c

            lax.fori_loop(0, SEQ, scale_row, 0, unroll=8)

        for b in range(NBUF - 1):
            fire(b, b)

        def ring_cycle(k, carry):
            for b in range(NBUF):
                ci = k * NBUF + b
                wait_gather(b)
                scale(b)
                store(ci, b)
                pb = (b - 1) % NBUF
                @pl.when(ci >= 1)
                def _():
                    wait_store(pb)
                @pl.when(ci + NBUF - 1 < rows_per_w)
                def _():
                    fire(ci + NBUF - 1, pb)
            return carry

        lax.fori_loop(0, rows_per_w // NBUF, ring_cycle, 0)
        wait_store((rows_per_w - 1) % NBUF)

    return pl.kernel(
        body,
        out_type=jax.ShapeDtypeStruct((nbatch, seq, EMB), jnp.float32),
        mesh=mesh,
        scratch_types=[
            pltpu.VMEM((toks_per_w,), jnp.int32),
            [pltpu.VMEM((SEQ, EMB), jnp.float32) for _ in range(NBUF)],
            [pltpu.SemaphoreType.DMA for _ in range(NBUF)],
            [pltpu.SemaphoreType.DMA for _ in range(NBUF)],
        ],
        compiler_params=pltpu.CompilerParams(use_tc_tiling_on_sc=False),
    )(tokens.reshape(B), table)
